# Initial kernel scaffold; baseline (speedup 1.0000x reference)
import functools

import jax
import jax.numpy as jnp
from jax import lax
from jax.experimental import pallas as pl
from jax.experimental.pallas import tpu as pltpu
from jax.experimental.pallas import tpu_sc as plsc

N = 10000
E = 320000
D = 128

NC = 2
NS = 16
L = 16

CHUNK = 128
EPT = 20096
E_PAD = EPT * NS
N_ITER = EPT // CHUNK
N_ACC = 10240
DEG_W = 16
ROWS_PER_TILE = N // NS


def _sc_body(edges_hbm, x_hbm, agg_out, deg_out,
             agg_sh, deg_sh, src_v, dst_v, rows_v, ones_v, z16_v, sem):
    c = lax.axis_index("c")
    s = lax.axis_index("s")

    def fill_row(i, _):
        for j in range(D // L):
            rows_v[i, pl.ds(j * L, L)] = jnp.zeros((L,), jnp.float32)
        ones_v[i, :] = jnp.ones((L,), jnp.float32)
        z16_v[i, :] = jnp.zeros((L,), jnp.float32)
        return 0
    lax.fori_loop(0, CHUNK, fill_row, 0)

    stripe = N_ACC // NS
    for k in range(stripe // CHUNK):
        base = s * stripe + k * CHUNK
        pltpu.sync_copy(rows_v, agg_sh.at[pl.ds(base, CHUNK)])
        pltpu.sync_copy(z16_v, deg_sh.at[pl.ds(base, CHUNK)])
    plsc.subcore_barrier()

    src_base = (2 * c) * E_PAD + s * EPT
    dst_base = (2 * c + 1) * E_PAD + s * EPT

    def step(i, _):
        off = i * CHUNK
        pltpu.sync_copy(edges_hbm.at[pl.ds(src_base + off, CHUNK)], src_v)
        pltpu.sync_copy(edges_hbm.at[pl.ds(dst_base + off, CHUNK)], dst_v)
        pltpu.async_copy(x_hbm.at[src_v], rows_v, sem).wait()
        pltpu.sync_copy(rows_v, agg_sh.at[dst_v], add=True)
        pltpu.sync_copy(ones_v, deg_sh.at[dst_v], add=True)
        return 0
    lax.fori_loop(0, N_ITER, step, 0)

    plsc.subcore_barrier()

    o = c * N + s * ROWS_PER_TILE
    pltpu.sync_copy(agg_sh.at[pl.ds(s * ROWS_PER_TILE, ROWS_PER_TILE)],
                    agg_out.at[pl.ds(o, ROWS_PER_TILE)])
    pltpu.sync_copy(deg_sh.at[pl.ds(s * ROWS_PER_TILE, ROWS_PER_TILE)],
                    deg_out.at[pl.ds(o, ROWS_PER_TILE)])


def _sc_aggregate(edges_flat, x):
    mesh = plsc.VectorSubcoreMesh(core_axis_name="c", subcore_axis_name="s",
                                  num_cores=NC, num_subcores=NS)
    return pl.kernel(
        _sc_body,
        out_type=(jax.ShapeDtypeStruct((2 * N, D), jnp.float32),
                  jax.ShapeDtypeStruct((2 * N, DEG_W), jnp.float32)),
        mesh=mesh,
        scratch_types=[
            pltpu.VMEM_SHARED((N_ACC, D), jnp.float32),
            pltpu.VMEM_SHARED((N_ACC, DEG_W), jnp.float32),
            pltpu.VMEM((CHUNK,), jnp.int32),
            pltpu.VMEM((CHUNK,), jnp.int32),
            pltpu.VMEM((CHUNK, D), jnp.float32),
            pltpu.VMEM((CHUNK, DEG_W), jnp.float32),
            pltpu.VMEM((CHUNK, DEG_W), jnp.float32),
            pltpu.SemaphoreType.DMA,
        ],
    )(edges_flat, x)


def _tc_body(af, ab, df, db, xb, wlf, wlb, wrf, wrb, bf, bb, out):
    mean_f = af[...] / jnp.maximum(df[...][:, 0:1], 1.0)
    mean_b = ab[...] / jnp.maximum(db[...][:, 0:1], 1.0)
    dn = (((1,), (1,)), ((), ()))
    z = lax.dot_general(mean_f, wlf[...], dn, preferred_element_type=jnp.float32)
    z = z + lax.dot_general(mean_b, wlb[...], dn, preferred_element_type=jnp.float32)
    z = z + lax.dot_general(xb[...], wrf[...] + wrb[...], dn,
                            preferred_element_type=jnp.float32)
    z = z + (bf[...] + bb[...])
    out[...] = jnp.maximum(0.5 * z, 0.0)


def _tc_tail(agg2, deg2, x, Wl_f, Wl_b, Wr_f, Wr_b, bl_f, bl_b):
    B = 1000
    grid = (N // B,)
    row = lambda i: (i, 0)
    row_off = lambda i: (i + N // B, 0)
    full = lambda i: (0, 0)
    return pl.pallas_call(
        _tc_body,
        grid=grid,
        in_specs=[
            pl.BlockSpec((B, D), row),
            pl.BlockSpec((B, D), row_off),
            pl.BlockSpec((B, DEG_W), row),
            pl.BlockSpec((B, DEG_W), row_off),
            pl.BlockSpec((B, D), row),
            pl.BlockSpec((D, D), full),
            pl.BlockSpec((D, D), full),
            pl.BlockSpec((D, D), full),
            pl.BlockSpec((D, D), full),
            pl.BlockSpec((1, D), full),
            pl.BlockSpec((1, D), full),
        ],
        out_specs=pl.BlockSpec((B, D), row),
        out_shape=jax.ShapeDtypeStruct((N, D), jnp.float32),
    )(agg2, deg2, x, Wl_f, Wl_b, Wr_f, Wr_b, bl_f, bl_b)


@jax.jit
def kernel(x, edge_index, reverse_edge_index, Wl_f, bl_f, Wr_f, Wl_b, bl_b, Wr_b):
    ei = edge_index.astype(jnp.int32)
    rei = reverse_edge_index.astype(jnp.int32)
    pad = E_PAD - E
    pad_src = jnp.zeros((pad,), jnp.int32)
    pad_dst = jnp.full((pad,), N, jnp.int32)
    edges_flat = jnp.concatenate([
        ei[0], pad_src, ei[1], pad_dst,
        rei[0], pad_src, rei[1], pad_dst,
    ])
    agg2, deg2 = _sc_aggregate(edges_flat, x)
    return _tc_tail(agg2, deg2, x,
                    Wl_f, Wl_b, Wr_f, Wr_b,
                    bl_f.reshape(1, D), bl_b.reshape(1, D))


# speedup vs baseline: 5.1150x; 5.1150x over previous
import jax
import jax.numpy as jnp
from jax import lax
from jax.experimental import pallas as pl
from jax.experimental.pallas import tpu as pltpu
from jax.experimental.pallas import tpu_sc as plsc

N = 10000
E = 320000
D = 128

NC = 2
NS = 16
L = 16

CHUNK = 128
EPT = 20096
E_PAD = EPT * NS
N_ITER = EPT // CHUNK
N_ACC = 10240

EB = 512
NEB = E // EB


def _sc_body(edges_hbm, x_hbm, agg_out, agg_sh, src_v, dst_v, rows_v, sem):
    c = lax.axis_index("c")
    s = lax.axis_index("s")

    def fill_row(i, _):
        for j in range(D // L):
            rows_v[i, pl.ds(j * L, L)] = jnp.zeros((L,), jnp.float32)
        return 0
    lax.fori_loop(0, CHUNK, fill_row, 0)

    stripe = N_ACC // NS
    for k in range(stripe // CHUNK):
        pltpu.sync_copy(rows_v, agg_sh.at[pl.ds(s * stripe + k * CHUNK, CHUNK)])
    plsc.subcore_barrier()

    src_base = (2 * c) * E_PAD + s * EPT
    dst_base = (2 * c + 1) * E_PAD + s * EPT

    def step(i, _):
        off = i * CHUNK
        pltpu.sync_copy(edges_hbm.at[pl.ds(src_base + off, CHUNK)], src_v)
        pltpu.sync_copy(edges_hbm.at[pl.ds(dst_base + off, CHUNK)], dst_v)
        pltpu.async_copy(x_hbm.at[src_v], rows_v, sem).wait()
        pltpu.sync_copy(rows_v, agg_sh.at[dst_v], add=True)
        return 0
    lax.fori_loop(0, N_ITER, step, 0)

    plsc.subcore_barrier()

    o = c * N_ACC + s * stripe
    pltpu.sync_copy(agg_sh.at[pl.ds(s * stripe, stripe)],
                    agg_out.at[pl.ds(o, stripe)])


def _sc_aggregate(edges_flat, x):
    mesh = plsc.VectorSubcoreMesh(core_axis_name="c", subcore_axis_name="s",
                                  num_cores=NC, num_subcores=NS)
    return pl.kernel(
        _sc_body,
        out_type=jax.ShapeDtypeStruct((2 * N_ACC, D), jnp.float32),
        mesh=mesh,
        scratch_types=[
            pltpu.VMEM_SHARED((N_ACC, D), jnp.float32),
            pltpu.VMEM((CHUNK,), jnp.int32),
            pltpu.VMEM((CHUNK,), jnp.int32),
            pltpu.VMEM((CHUNK, D), jnp.float32),
            pltpu.SemaphoreType.DMA,
        ],
    )(edges_flat, x)


def _deg_body(dst_ref, out_ref):
    i = pl.program_id(1)
    d = dst_ref[0]
    hi = d >> 7
    lo = d & 127
    rows = lax.broadcasted_iota(jnp.int32, (D, EB), 0)
    hi_t = (rows == hi).astype(jnp.bfloat16)
    lo_t = (rows == lo).astype(jnp.bfloat16)
    dn = (((1,), (1,)), ((), ()))
    grid = lax.dot_general(hi_t, lo_t, dn, preferred_element_type=jnp.float32)

    @pl.when(i == 0)
    def _():
        out_ref[0] = jnp.zeros((D, D), jnp.float32)
    out_ref[0] += grid


def _tc_degrees(dst2):
    return pl.pallas_call(
        _deg_body,
        grid=(2, NEB),
        in_specs=[pl.BlockSpec((1, 1, EB), lambda c, i: (c * NEB + i, 0, 0))],
        out_specs=pl.BlockSpec((1, D, D), lambda c, i: (c, 0, 0)),
        out_shape=jax.ShapeDtypeStruct((2, D, D), jnp.float32),
    )(dst2)


def _tc_body(af, ab, df, db, xb, wlf, wlb, wrf, wrb, bf, bb, out):
    mean_f = af[...] / jnp.maximum(df[...], 1.0)
    mean_b = ab[...] / jnp.maximum(db[...], 1.0)
    dn = (((1,), (1,)), ((), ()))
    z = lax.dot_general(mean_f, wlf[...], dn, preferred_element_type=jnp.float32)
    z = z + lax.dot_general(mean_b, wlb[...], dn, preferred_element_type=jnp.float32)
    z = z + lax.dot_general(xb[...], wrf[...] + wrb[...], dn,
                            preferred_element_type=jnp.float32)
    z = z + (bf[...] + bb[...])
    out[...] = jnp.maximum(0.5 * z, 0.0)


def _tc_tail(af, ab, df, db, x, Wl_f, Wl_b, Wr_f, Wr_b, bl_f, bl_b):
    B = 1000
    grid = (N // B,)
    row = lambda i: (i, 0)
    full = lambda i: (0, 0)
    return pl.pallas_call(
        _tc_body,
        grid=grid,
        in_specs=[
            pl.BlockSpec((B, D), row),
            pl.BlockSpec((B, D), row),
            pl.BlockSpec((B, 1), row),
            pl.BlockSpec((B, 1), row),
            pl.BlockSpec((B, D), row),
            pl.BlockSpec((D, D), full),
            pl.BlockSpec((D, D), full),
            pl.BlockSpec((D, D), full),
            pl.BlockSpec((D, D), full),
            pl.BlockSpec((1, D), full),
            pl.BlockSpec((1, D), full),
        ],
        out_specs=pl.BlockSpec((B, D), row),
        out_shape=jax.ShapeDtypeStruct((N, D), jnp.float32),
    )(af, ab, df, db, x, Wl_f, Wl_b, Wr_f, Wr_b, bl_f, bl_b)


@jax.jit
def kernel(x, edge_index, reverse_edge_index, Wl_f, bl_f, Wr_f, Wl_b, bl_b, Wr_b):
    ei = edge_index.astype(jnp.int32)
    rei = reverse_edge_index.astype(jnp.int32)
    pad = E_PAD - E
    pad_src = jnp.zeros((pad,), jnp.int32)
    pad_dst = jnp.full((pad,), N, jnp.int32)
    edges_flat = jnp.concatenate([
        ei[0], pad_src, ei[1], pad_dst,
        rei[0], pad_src, rei[1], pad_dst,
    ])
    dst2 = jnp.concatenate([ei[1], rei[1]]).reshape(2 * NEB, 1, EB)

    agg2 = _sc_aggregate(edges_flat, x)
    deg_hl = _tc_degrees(dst2)

    af = lax.slice(agg2, (0, 0), (N, D))
    ab = lax.slice(agg2, (N_ACC, 0), (N_ACC + N, D))
    df = deg_hl[0].reshape(D * D)[:N, None]
    db = deg_hl[1].reshape(D * D)[:N, None]
    return _tc_tail(af, ab, df, db, x,
                    Wl_f, Wl_b, Wr_f, Wr_b,
                    bl_f.reshape(1, D), bl_b.reshape(1, D))


# speedup vs baseline: 5.4789x; 1.0711x over previous
import jax
import jax.numpy as jnp
from jax import lax
from jax.experimental import pallas as pl
from jax.experimental.pallas import tpu as pltpu
from jax.experimental.pallas import tpu_sc as plsc

N = 10000
E = 320000
D = 128

NC = 2
NS = 16
L = 16

CHUNK = 128
N_ITER = 157
EPT = N_ITER * CHUNK
E_PAD = EPT * NS
N_ACC = 10240

EB = 512
NEB = E // EB


def _sc_body(edges_hbm, x_hbm, agg_out, agg_sh,
             idx0, idx1, rows0, rows1, si0, si1, sg0, sg1):
    c = lax.axis_index("c")
    s = lax.axis_index("s")
    idx_v = (idx0, idx1)
    rows_v = (rows0, rows1)
    sem_i = (si0, si1)
    sem_g = (sg0, sg1)

    def fill_row(i, _):
        for j in range(D // L):
            rows0[i, pl.ds(j * L, L)] = jnp.zeros((L,), jnp.float32)
        return 0
    lax.fori_loop(0, CHUNK, fill_row, 0)

    stripe = N_ACC // NS
    for k in range(stripe // CHUNK):
        pltpu.sync_copy(rows0, agg_sh.at[pl.ds(s * stripe + k * CHUNK, CHUNK)])
    plsc.subcore_barrier()

    base_blk = (c * NS + s) * N_ITER

    pltpu.sync_copy(edges_hbm.at[base_blk], idx0)
    pltpu.async_copy(edges_hbm.at[base_blk + 1], idx1, si1)
    pltpu.async_copy(x_hbm.at[idx0.at[0]], rows0, sg0)

    def pair(g, _):
        for b in range(2):
            i = 2 * g + b
            o = 1 - b
            pltpu.make_async_copy(x_hbm.at[idx_v[b].at[0]], rows_v[b],
                                  sem_g[b]).wait()
            pltpu.sync_copy(rows_v[b], agg_sh.at[idx_v[b].at[1]], add=True)
            nb = jnp.minimum(i + 2, N_ITER - 1)
            pltpu.async_copy(edges_hbm.at[base_blk + nb], idx_v[b], sem_i[b])
            pltpu.make_async_copy(edges_hbm.at[base_blk], idx_v[o],
                                  sem_i[o]).wait()
            pltpu.async_copy(x_hbm.at[idx_v[o].at[0]], rows_v[o], sem_g[o])
        return 0
    lax.fori_loop(0, (N_ITER - 1) // 2, pair, 0)

    pltpu.make_async_copy(edges_hbm.at[base_blk], idx1, si1).wait()
    pltpu.make_async_copy(x_hbm.at[idx0.at[0]], rows0, sg0).wait()
    pltpu.sync_copy(rows0, agg_sh.at[idx0.at[1]], add=True)

    plsc.subcore_barrier()

    o = c * N_ACC + s * stripe
    pltpu.sync_copy(agg_sh.at[pl.ds(s * stripe, stripe)],
                    agg_out.at[pl.ds(o, stripe)])


def _sc_aggregate(edges_blk, x):
    mesh = plsc.VectorSubcoreMesh(core_axis_name="c", subcore_axis_name="s",
                                  num_cores=NC, num_subcores=NS)
    return pl.kernel(
        _sc_body,
        out_type=jax.ShapeDtypeStruct((2 * N_ACC, D), jnp.float32),
        mesh=mesh,
        scratch_types=[
            pltpu.VMEM_SHARED((N_ACC, D), jnp.float32),
            pltpu.VMEM((2, CHUNK), jnp.int32),
            pltpu.VMEM((2, CHUNK), jnp.int32),
            pltpu.VMEM((CHUNK, D), jnp.float32),
            pltpu.VMEM((CHUNK, D), jnp.float32),
            pltpu.SemaphoreType.DMA,
            pltpu.SemaphoreType.DMA,
            pltpu.SemaphoreType.DMA,
            pltpu.SemaphoreType.DMA,
        ],
    )(edges_blk, x)


def _deg_body(dst_ref, out_ref):
    i = pl.program_id(1)
    d = dst_ref[0]
    hi = d >> 7
    lo = d & 127
    rows = lax.broadcasted_iota(jnp.int32, (D, EB), 0)
    hi_t = (rows == hi).astype(jnp.bfloat16)
    lo_t = (rows == lo).astype(jnp.bfloat16)
    dn = (((1,), (1,)), ((), ()))
    grid = lax.dot_general(hi_t, lo_t, dn, preferred_element_type=jnp.float32)

    @pl.when(i == 0)
    def _():
        out_ref[0] = jnp.zeros((D, D), jnp.float32)
    out_ref[0] += grid


def _tc_degrees(dst2):
    return pl.pallas_call(
        _deg_body,
        grid=(2, NEB),
        in_specs=[pl.BlockSpec((1, 1, EB), lambda c, i: (c * NEB + i, 0, 0))],
        out_specs=pl.BlockSpec((1, D, D), lambda c, i: (c, 0, 0)),
        out_shape=jax.ShapeDtypeStruct((2, D, D), jnp.float32),
    )(dst2)


def _tc_body(af, ab, df, db, xb, wlf, wlb, wrf, wrb, bf, bb, out):
    mean_f = af[...] / jnp.maximum(df[...], 1.0)
    mean_b = ab[...] / jnp.maximum(db[...], 1.0)
    dn = (((1,), (1,)), ((), ()))
    z = lax.dot_general(mean_f, wlf[...], dn, preferred_element_type=jnp.float32)
    z = z + lax.dot_general(mean_b, wlb[...], dn, preferred_element_type=jnp.float32)
    z = z + lax.dot_general(xb[...], wrf[...] + wrb[...], dn,
                            preferred_element_type=jnp.float32)
    z = z + (bf[...] + bb[...])
    out[...] = jnp.maximum(0.5 * z, 0.0)


def _tc_tail(af, ab, df, db, x, Wl_f, Wl_b, Wr_f, Wr_b, bl_f, bl_b):
    B = 1000
    grid = (N // B,)
    row = lambda i: (i, 0)
    full = lambda i: (0, 0)
    return pl.pallas_call(
        _tc_body,
        grid=grid,
        in_specs=[
            pl.BlockSpec((B, D), row),
            pl.BlockSpec((B, D), row),
            pl.BlockSpec((B, 1), row),
            pl.BlockSpec((B, 1), row),
            pl.BlockSpec((B, D), row),
            pl.BlockSpec((D, D), full),
            pl.BlockSpec((D, D), full),
            pl.BlockSpec((D, D), full),
            pl.BlockSpec((D, D), full),
            pl.BlockSpec((1, D), full),
            pl.BlockSpec((1, D), full),
        ],
        out_specs=pl.BlockSpec((B, D), row),
        out_shape=jax.ShapeDtypeStruct((N, D), jnp.float32),
    )(af, ab, df, db, x, Wl_f, Wl_b, Wr_f, Wr_b, bl_f, bl_b)


@jax.jit
def kernel(x, edge_index, reverse_edge_index, Wl_f, bl_f, Wr_f, Wl_b, bl_b, Wr_b):
    ei = edge_index.astype(jnp.int32)
    rei = reverse_edge_index.astype(jnp.int32)
    pad = E_PAD - E
    pad_src = jnp.zeros((pad,), jnp.int32)
    pad_dst = jnp.full((pad,), N, jnp.int32)

    def blocks(src, dst):
        s = jnp.concatenate([src, pad_src]).reshape(NS, N_ITER, 1, CHUNK)
        d = jnp.concatenate([dst, pad_dst]).reshape(NS, N_ITER, 1, CHUNK)
        return jnp.concatenate([s, d], axis=2)

    edges_blk = jnp.concatenate(
        [blocks(ei[0], ei[1]), blocks(rei[0], rei[1])]
    ).reshape(NC * NS * N_ITER, 2, CHUNK)
    dst2 = jnp.concatenate([ei[1], rei[1]]).reshape(2 * NEB, 1, EB)

    agg2 = _sc_aggregate(edges_blk, x)
    deg_hl = _tc_degrees(dst2)

    af = lax.slice(agg2, (0, 0), (N, D))
    ab = lax.slice(agg2, (N_ACC, 0), (N_ACC + N, D))
    df = deg_hl[0].reshape(D * D)[:N, None]
    db = deg_hl[1].reshape(D * D)[:N, None]
    return _tc_tail(af, ab, df, db, x,
                    Wl_f, Wl_b, Wr_f, Wr_b,
                    bl_f.reshape(1, D), bl_b.reshape(1, D))
